# baseline (speedup 1.0000x reference)
import functools

import jax
import jax.numpy as jnp
from jax import lax
from jax.experimental import pallas as pl
from jax.experimental.pallas import tpu as pltpu
from jax.experimental.pallas import tpu_sc as plsc

_NUM_CORES = 2
_NUM_SUBCORES = 16
_NW = _NUM_CORES * _NUM_SUBCORES

_T = 16384
_S = 50
_D = 32
_CH = 1024
_NTG = _T // _CH
_NCHUNK = _S * _NTG
_JPW = _NCHUNK // _NW

_mesh = plsc.VectorSubcoreMesh(core_axis_name="c", subcore_axis_name="s")


@functools.partial(
    pl.kernel,
    out_type=jax.ShapeDtypeStruct((_S, 4, 131072), jnp.float32),
    mesh=_mesh,
    scratch_types=[
        pltpu.VMEM((_CH,), jnp.int32),
        pltpu.VMEM((_CH,), jnp.int32),
        pltpu.VMEM((_CH, _D), jnp.float32),
        pltpu.VMEM((_CH, _D), jnp.float32),
        pltpu.VMEM((4 * 8192,), jnp.float32),
        pltpu.SemaphoreType.DMA,
        pltpu.SemaphoreType.DMA,
        pltpu.SemaphoreType.DMA,
        pltpu.SemaphoreType.DMA,
        pltpu.SemaphoreType.DMA,
    ],
    compiler_params=pltpu.CompilerParams(
        use_tc_tiling_on_sc=False, needs_layout_passes=False),
)
def _gather_kernel(tok_hbm, table_hbm, out_hbm, idx0, idx1, rows0, rows1,
                   tr, si0, si1, sg0, sg1, st):
    wid = lax.axis_index("s") * _NUM_CORES + lax.axis_index("c")
    c0 = wid * _JPW

    idx = (idx0, idx1)
    rows = (rows0, rows1)
    si = (si0, si1)
    sg = (sg0, sg1)
    iota = lax.iota(jnp.int32, 16)

    cols = {}
    dsts = {}
    for c_half in (0, 16):
        for k in range(16):
            rot = (iota + k) % 16
            dg = c_half + rot
            cols[(c_half, k)] = dg
            dsts[(c_half, k)] = (dg // 8) * 8192 + (dg % 8) * 128 + iota

    def s_tg(j):
        c = c0 + j
        return c // _NTG, c % _NTG

    def idx_cp(j, b):
        s, tg = s_tg(j)
        return pltpu.make_async_copy(tok_hbm.at[s, tg], idx[b], si[b])

    def gather_cp(j, b):
        return pltpu.make_async_copy(table_hbm.at[idx[b]], rows[b], sg[b])

    def store_cp(j, dt):
        s, tg = s_tg(j)
        return pltpu.make_async_copy(
            tr.at[pl.ds(dt * 8192, 8192)],
            out_hbm.at[s, dt, pl.ds(tg * 8192, 8192)], st)

    def transpose_chunk(b):
        @plsc.parallel_loop(0, 64, unroll=2)
        def t_body(t16):
            t0 = t16 * 16
            tq = t0 // 128
            sbase = tq * 1024 + (t0 - tq * 128)
            r_idx = t0 + iota
            for c_half in (0, 16):
                for k in range(16):
                    v = plsc.load_gather(rows[b], [r_idx, cols[(c_half, k)]])
                    plsc.store_scatter(tr, [dsts[(c_half, k)] + sbase], v)

    idx_cp(0, 0).start()
    idx_cp(1, 1).start()
    idx_cp(0, 0).wait()
    gather_cp(0, 0).start()

    def step(j, b):
        o = 1 - b

        @pl.when(j + 1 < _JPW)
        def _():
            idx_cp(j + 1, o).wait()
            gather_cp(j + 1, o).start()

        gather_cp(j, b).wait()

        @pl.when(j + 2 < _JPW)
        def _():
            idx_cp(j + 2, b).start()

        @pl.when(j >= 1)
        def _():
            for dt in range(4):
                store_cp(j - 1, dt).wait()

        transpose_chunk(b)
        for dt in range(4):
            store_cp(j, dt).start()

    @pl.loop(0, _JPW - 1, step=2)
    def step2(j0):
        for db in range(2):
            step(j0 + db, db)

    step(_JPW - 1, (_JPW - 1) % 2)
    for dt in range(4):
        store_cp(_JPW - 1, dt).wait()


def kernel(token_ids, weight):
    tok3 = jnp.transpose(token_ids).reshape(_S, _NTG, _CH).astype(jnp.int32)
    out3 = _gather_kernel(tok3, weight)
    out5 = out3.reshape(_S, 4, 128, 8, 128)
    return out5.transpose(2, 4, 0, 1, 3).reshape(_T, _S, _D)


# speedup vs baseline: 1.0352x; 1.0352x over previous
import functools

import jax
import jax.numpy as jnp
from jax import lax
from jax.experimental import pallas as pl
from jax.experimental.pallas import tpu as pltpu
from jax.experimental.pallas import tpu_sc as plsc

_NUM_CORES = 2
_NUM_SUBCORES = 16
_NW = _NUM_CORES * _NUM_SUBCORES

_T = 16384
_S = 50
_D = 32
_CH = 1024
_NTG = _T // _CH
_NCHUNK = _S * _NTG
_JPW = _NCHUNK // _NW

_mesh = plsc.VectorSubcoreMesh(core_axis_name="c", subcore_axis_name="s")


@functools.partial(
    pl.kernel,
    out_type=jax.ShapeDtypeStruct((_S, 4, 131072), jnp.float32),
    mesh=_mesh,
    scratch_types=[
        pltpu.VMEM((_CH,), jnp.int32),
        pltpu.VMEM((_CH,), jnp.int32),
        pltpu.VMEM((_CH, _D), jnp.float32),
        pltpu.VMEM((_CH, _D), jnp.float32),
        pltpu.VMEM((4 * 8192,), jnp.float32),
        pltpu.SemaphoreType.DMA,
        pltpu.SemaphoreType.DMA,
        pltpu.SemaphoreType.DMA,
        pltpu.SemaphoreType.DMA,
        pltpu.SemaphoreType.DMA,
    ],
    compiler_params=pltpu.CompilerParams(
        use_tc_tiling_on_sc=False, needs_layout_passes=False),
)
def _gather_kernel(tok_hbm, table_hbm, out_hbm, idx0, idx1, rows0, rows1,
                   tr, si0, si1, sg0, sg1, st):
    wid = lax.axis_index("s") * _NUM_CORES + lax.axis_index("c")
    c0 = wid * _JPW

    idx = (idx0, idx1)
    rows = (rows0, rows1)
    si = (si0, si1)
    sg = (sg0, sg1)
    iota = lax.iota(jnp.int32, 16)

    cols = {}
    dsts = {}
    for c_half in (0, 16):
        for k in range(16):
            rot = (iota + k) % 16
            dg = c_half + rot
            cols[(c_half, k)] = dg
            dsts[(c_half, k)] = (dg // 8) * 8192 + (dg % 8) * 128 + iota

    def s_tg(j):
        c = c0 + j
        return c // _NTG, c % _NTG

    def idx_cp(j, b):
        s, tg = s_tg(j)
        return pltpu.make_async_copy(tok_hbm.at[s, tg], idx[b], si[b])

    def gather_cp(j, b):
        return pltpu.make_async_copy(table_hbm.at[idx[b]], rows[b], sg[b])

    def store_cp(j, dt):
        s, tg = s_tg(j)
        return pltpu.make_async_copy(
            tr.at[pl.ds(dt * 8192, 8192)],
            out_hbm.at[s, dt, pl.ds(tg * 8192, 8192)], st)

    def transpose_chunk(b):
        @plsc.parallel_loop(0, 64)
        def t_body(t16):
            t0 = t16 * 16
            tq = t0 // 128
            sbase = tq * 1024 + (t0 - tq * 128)
            r_idx = t0 + iota
            for c_half in (0, 16):
                for k in range(16):
                    v = plsc.load_gather(rows[b], [r_idx, cols[(c_half, k)]])
                    plsc.store_scatter(tr, [dsts[(c_half, k)] + sbase], v)

    idx_cp(0, 0).start()
    idx_cp(1, 1).start()
    idx_cp(0, 0).wait()
    gather_cp(0, 0).start()

    def step(j, b):
        o = 1 - b

        @pl.when(j + 1 < _JPW)
        def _():
            idx_cp(j + 1, o).wait()
            gather_cp(j + 1, o).start()

        gather_cp(j, b).wait()

        @pl.when(j + 2 < _JPW)
        def _():
            idx_cp(j + 2, b).start()

        @pl.when(j >= 1)
        def _():
            for dt in range(4):
                store_cp(j - 1, dt).wait()

        transpose_chunk(b)
        for dt in range(4):
            store_cp(j, dt).start()

    @pl.loop(0, _JPW - 1, step=2)
    def step2(j0):
        for db in range(2):
            step(j0 + db, db)

    step(_JPW - 1, (_JPW - 1) % 2)
    for dt in range(4):
        store_cp(_JPW - 1, dt).wait()


def kernel(token_ids, weight):
    tok3 = jnp.transpose(token_ids).reshape(_S, _NTG, _CH).astype(jnp.int32)
    out3 = _gather_kernel(tok3, weight)
    out5 = out3.reshape(_S, 4, 128, 8, 128)
    return out5.transpose(2, 4, 0, 1, 3).reshape(_T, _S, _D)
